# initial kernel scaffold (unmeasured)
import jax
import jax.numpy as jnp
from jax import lax
from jax.experimental import pallas as pl
from jax.experimental.pallas import tpu as pltpu

B, H, D, BS = 8, 8, 128, 16
NB = 512
NLOCAL = 512
CHUNK = 8
SCALE = D ** -0.5
NEG = -1e30


def kernel(Q, K, V, bt, lens):
    my_z = lax.axis_index("z")

    slot = jnp.arange(NB, dtype=jnp.int32)[None, :]
    valid = (slot < lens[:, None]) & ((bt // NLOCAL) == my_z)
    order = jnp.argsort(jnp.logical_not(valid), axis=1)
    pages = jnp.take_along_axis(bt % NLOCAL, order, axis=1).astype(jnp.int32)
    counts = jnp.sum(valid, axis=1).astype(jnp.int32)

    q = Q[:, 0]

    def body(q_ref, pages_ref, counts_ref, k_hbm, v_hbm, out_ref,
             kbuf, vbuf, ksem, vsem,
             acc_s, ml_s, racc, rml, ssem, rsem):
        my_x = lax.axis_index("x")
        my_y = lax.axis_index("y")
        mz = lax.axis_index("z")

        def start_fetch(i, step, buf):
            for c in range(CHUNK):
                p = pages_ref[i, step * CHUNK + c]
                pltpu.make_async_copy(
                    k_hbm.at[p], kbuf.at[buf, c], ksem.at[buf, c]).start()
                pltpu.make_async_copy(
                    v_hbm.at[p], vbuf.at[buf, c], vsem.at[buf, c]).start()

        def wait_fetch(buf):
            for c in range(CHUNK):
                pltpu.make_async_copy(
                    k_hbm.at[0], kbuf.at[buf, c], ksem.at[buf, c]).wait()
                pltpu.make_async_copy(
                    v_hbm.at[0], vbuf.at[buf, c], vsem.at[buf, c]).wait()

        for i in range(B):
            n = counts_ref[i]
            steps = (n + CHUNK - 1) // CHUNK
            qi = q_ref[i]

            @pl.when(steps > 0)
            def _():
                start_fetch(i, 0, 0)

            def step_fn(s, carry):
                m, l, acc = carry
                buf = lax.rem(s, 2)

                @pl.when(s + 1 < steps)
                def _():
                    start_fetch(i, s + 1, lax.rem(s + 1, 2))

                wait_fetch(buf)
                kc = kbuf[buf].reshape(CHUNK * BS, H, D)
                vc = vbuf[buf].reshape(CHUNK * BS, H, D)
                sc = jnp.einsum("hd,khd->hk", qi, kc,
                                preferred_element_type=jnp.float32) * SCALE
                tok = lax.broadcasted_iota(jnp.int32, (1, CHUNK * BS), 1)
                key_ok = (s * CHUNK + tok // BS) < n
                sc = jnp.where(key_ok, sc, NEG)
                m_new = jnp.maximum(m, jnp.max(sc, axis=-1, keepdims=True))
                p = jnp.exp(sc - m_new)
                corr = jnp.exp(m - m_new)
                l_new = l * corr + jnp.sum(p, axis=-1, keepdims=True)
                pv = jnp.einsum("hk,khd->hd", p, vc,
                                preferred_element_type=jnp.float32)
                acc_new = acc * corr + pv
                return m_new, l_new, acc_new

            m0 = jnp.full((H, 1), NEG, dtype=jnp.float32)
            l0 = jnp.zeros((H, 1), dtype=jnp.float32)
            a0 = jnp.zeros((H, D), dtype=jnp.float32)
            m, l, acc = lax.fori_loop(0, steps, step_fn, (m0, l0, a0))

            acc_s[i] = acc
            ml_s[0, i] = m[:, 0]
            ml_s[1, i] = l[:, 0]

        partner = (my_x, my_y, 1 - mz)
        barrier = pltpu.get_barrier_semaphore()
        pl.semaphore_signal(barrier, inc=1, device_id=partner,
                            device_id_type=pl.DeviceIdType.MESH)
        pl.semaphore_wait(barrier, 1)

        r_acc = pltpu.make_async_remote_copy(
            src_ref=acc_s, dst_ref=racc, send_sem=ssem.at[0],
            recv_sem=rsem.at[0], device_id=partner,
            device_id_type=pl.DeviceIdType.MESH)
        r_ml = pltpu.make_async_remote_copy(
            src_ref=ml_s, dst_ref=rml, send_sem=ssem.at[1],
            recv_sem=rsem.at[1], device_id=partner,
            device_id_type=pl.DeviceIdType.MESH)
        r_acc.start()
        r_ml.start()
        r_acc.wait()
        r_ml.wait()

        m_a, l_a = ml_s[0], ml_s[1]
        m_b, l_b = rml[0], rml[1]
        m = jnp.maximum(m_a, m_b)
        wa = jnp.exp(m_a - m)
        wb = jnp.exp(m_b - m)
        l = wa * l_a + wb * l_b
        acc = (wa[:, :, None] * acc_s[...] + wb[:, :, None] * racc[...])
        out_ref[...] = (acc / l[:, :, None]).reshape(B, 1, H, D)

    return pl.pallas_call(
        body,
        out_shape=jax.ShapeDtypeStruct((B, 1, H, D), jnp.float32),
        in_specs=[
            pl.BlockSpec(memory_space=pltpu.VMEM),
            pl.BlockSpec(memory_space=pltpu.SMEM),
            pl.BlockSpec(memory_space=pltpu.SMEM),
            pl.BlockSpec(memory_space=pltpu.ANY),
            pl.BlockSpec(memory_space=pltpu.ANY),
        ],
        out_specs=pl.BlockSpec(memory_space=pltpu.VMEM),
        scratch_shapes=[
            pltpu.VMEM((2, CHUNK, BS, H, D), jnp.float32),
            pltpu.VMEM((2, CHUNK, BS, H, D), jnp.float32),
            pltpu.SemaphoreType.DMA((2, CHUNK)),
            pltpu.SemaphoreType.DMA((2, CHUNK)),
            pltpu.VMEM((B, H, D), jnp.float32),
            pltpu.VMEM((2, B, H), jnp.float32),
            pltpu.VMEM((B, H, D), jnp.float32),
            pltpu.VMEM((2, B, H), jnp.float32),
            pltpu.SemaphoreType.DMA((2,)),
            pltpu.SemaphoreType.DMA((2,)),
        ],
        compiler_params=pltpu.CompilerParams(collective_id=0),
    )(q, pages, counts, K, V)


# baseline (device time: 528831 ns/iter reference)
import jax
import jax.numpy as jnp
from jax import lax
from jax.experimental import pallas as pl
from jax.experimental.pallas import tpu as pltpu

B, H, D, BS = 8, 8, 128, 16
NB = 512
NLOCAL = 512
CHUNK = 8
SCALE = D ** -0.5
NEG = -1e30


def kernel(Q, K, V, bt, lens):
    my_z = lax.axis_index("z")

    slot = jnp.arange(NB, dtype=jnp.int32)[None, :]
    valid = (slot < lens[:, None]) & ((bt // NLOCAL) == my_z)
    order = jnp.argsort(jnp.logical_not(valid), axis=1)
    pages = jnp.take_along_axis(bt % NLOCAL, order, axis=1).astype(jnp.int32)
    counts = jnp.sum(valid, axis=1).astype(jnp.int32)

    q = Q[:, 0]

    def body(q_ref, pages_ref, counts_ref, k_hbm, v_hbm, out_ref,
             kbuf, vbuf, ksem, vsem,
             acc_s, ml_s, racc, rml, ssem, rsem):
        my_x = lax.axis_index("x")
        my_y = lax.axis_index("y")
        mz = lax.axis_index("z")

        def start_fetch(i, step, buf):
            for c in range(CHUNK):
                p = pages_ref[i, step * CHUNK + c]
                pltpu.make_async_copy(
                    k_hbm.at[p], kbuf.at[buf, c], ksem.at[buf, c]).start()
                pltpu.make_async_copy(
                    v_hbm.at[p], vbuf.at[buf, c], vsem.at[buf, c]).start()

        def wait_fetch(buf):
            for c in range(CHUNK):
                pltpu.make_async_copy(
                    k_hbm.at[0], kbuf.at[buf, c], ksem.at[buf, c]).wait()
                pltpu.make_async_copy(
                    v_hbm.at[0], vbuf.at[buf, c], vsem.at[buf, c]).wait()

        for i in range(B):
            n = counts_ref[i]
            steps = (n + CHUNK - 1) // CHUNK
            qi = q_ref[i]

            @pl.when(steps > 0)
            def _():
                start_fetch(i, 0, 0)

            def step_fn(s, carry):
                m, l, acc = carry
                buf = lax.rem(s, 2)

                @pl.when(s + 1 < steps)
                def _():
                    start_fetch(i, s + 1, lax.rem(s + 1, 2))

                wait_fetch(buf)
                kc = kbuf[buf].reshape(CHUNK * BS, H, D)
                vc = vbuf[buf].reshape(CHUNK * BS, H, D)
                sc = jnp.einsum("hd,khd->hk", qi, kc,
                                preferred_element_type=jnp.float32) * SCALE
                tok = lax.broadcasted_iota(jnp.int32, (1, CHUNK * BS), 1)
                key_ok = (s * CHUNK + tok // BS) < n
                sc = jnp.where(key_ok, sc, NEG)
                m_new = jnp.maximum(m, jnp.max(sc, axis=-1, keepdims=True))
                p = jnp.exp(sc - m_new)
                corr = jnp.exp(m - m_new)
                l_new = l * corr + jnp.sum(p, axis=-1, keepdims=True)
                pv = jnp.einsum("hk,khd->hd", p, vc,
                                preferred_element_type=jnp.float32)
                acc_new = acc * corr + pv
                return m_new, l_new, acc_new

            m0 = jnp.full((H, 1), NEG, dtype=jnp.float32)
            l0 = jnp.zeros((H, 1), dtype=jnp.float32)
            a0 = jnp.zeros((H, D), dtype=jnp.float32)
            m, l, acc = lax.fori_loop(0, steps, step_fn, (m0, l0, a0))

            acc_s[i] = acc
            ml_s[0, i] = m[:, 0]
            ml_s[1, i] = l[:, 0]

        partner = (my_x, my_y, 1 - mz)
        barrier = pltpu.get_barrier_semaphore()
        pl.semaphore_signal(barrier, inc=1, device_id=partner,
                            device_id_type=pl.DeviceIdType.MESH)
        pl.semaphore_wait(barrier, 1)

        r_acc = pltpu.make_async_remote_copy(
            src_ref=acc_s, dst_ref=racc, send_sem=ssem.at[0],
            recv_sem=rsem.at[0], device_id=partner,
            device_id_type=pl.DeviceIdType.MESH)
        r_ml = pltpu.make_async_remote_copy(
            src_ref=ml_s, dst_ref=rml, send_sem=ssem.at[1],
            recv_sem=rsem.at[1], device_id=partner,
            device_id_type=pl.DeviceIdType.MESH)
        r_acc.start()
        r_ml.start()
        r_acc.wait()
        r_ml.wait()

        m_a, l_a = ml_s[0], ml_s[1]
        m_b, l_b = rml[0], rml[1]
        m = jnp.maximum(m_a, m_b)
        wa = jnp.exp(m_a - m)
        wb = jnp.exp(m_b - m)
        l = wa * l_a + wb * l_b
        acc = (wa[:, :, None] * acc_s[...] + wb[:, :, None] * racc[...])
        out_ref[...] = (acc / l[:, :, None]).reshape(B, 1, H, D)

    return pl.pallas_call(
        body,
        out_shape=jax.ShapeDtypeStruct((B, 1, H, D), jnp.float32),
        in_specs=[
            pl.BlockSpec(memory_space=pltpu.VMEM),
            pl.BlockSpec(memory_space=pltpu.SMEM),
            pl.BlockSpec(memory_space=pltpu.SMEM),
            pl.BlockSpec(memory_space=pl.ANY),
            pl.BlockSpec(memory_space=pl.ANY),
        ],
        out_specs=pl.BlockSpec(memory_space=pltpu.VMEM),
        scratch_shapes=[
            pltpu.VMEM((2, CHUNK, BS, H, D), jnp.float32),
            pltpu.VMEM((2, CHUNK, BS, H, D), jnp.float32),
            pltpu.SemaphoreType.DMA((2, CHUNK)),
            pltpu.SemaphoreType.DMA((2, CHUNK)),
            pltpu.VMEM((B, H, D), jnp.float32),
            pltpu.VMEM((2, B, H), jnp.float32),
            pltpu.VMEM((B, H, D), jnp.float32),
            pltpu.VMEM((2, B, H), jnp.float32),
            pltpu.SemaphoreType.DMA((2,)),
            pltpu.SemaphoreType.DMA((2,)),
        ],
        compiler_params=pltpu.CompilerParams(collective_id=0),
    )(q, pages, counts, K, V)


# device time: 182568 ns/iter; 2.8966x vs baseline; 2.8966x over previous
import jax
import jax.numpy as jnp
from jax import lax
from jax.experimental import pallas as pl
from jax.experimental.pallas import tpu as pltpu

B, H, D, BS = 8, 8, 128, 16
NB = 512
NLOCAL = 512
CHUNK = 8
SCALE = D ** -0.5
NEG = -1e30


def kernel(Q, K, V, bt, lens):
    my_z = lax.axis_index("z")
    my_q = lax.axis_index("x") * 2 + lax.axis_index("y")

    slot = jnp.arange(NB, dtype=jnp.int32)[None, :]
    valid = (slot < lens[:, None]) & ((bt // NLOCAL) == my_z)
    valid = valid & (slot % 4 == my_q)
    order = jnp.argsort(jnp.logical_not(valid), axis=1)
    pages = jnp.take_along_axis(bt % NLOCAL, order, axis=1).astype(jnp.int32)
    counts = jnp.sum(valid, axis=1).astype(jnp.int32)

    q = Q[:, 0]

    def body(q_ref, pages_ref, counts_ref, k_hbm, v_hbm, out_ref,
             kbuf, vbuf, ksem, vsem,
             acc_s, ml_s, racc, rml, ssem, rsem):
        my_x = lax.axis_index("x")
        my_y = lax.axis_index("y")
        mz = lax.axis_index("z")

        def start_fetch(i, step, buf):
            for c in range(CHUNK):
                p = pages_ref[i, step * CHUNK + c]
                pltpu.make_async_copy(
                    k_hbm.at[p], kbuf.at[buf, c], ksem.at[buf, c]).start()
                pltpu.make_async_copy(
                    v_hbm.at[p], vbuf.at[buf, c], vsem.at[buf, c]).start()

        def wait_fetch(buf):
            for c in range(CHUNK):
                pltpu.make_async_copy(
                    k_hbm.at[0], kbuf.at[buf, c], ksem.at[buf, c]).wait()
                pltpu.make_async_copy(
                    v_hbm.at[0], vbuf.at[buf, c], vsem.at[buf, c]).wait()

        for i in range(B):
            n = counts_ref[i]
            steps = (n + CHUNK - 1) // CHUNK
            qi = q_ref[i]

            @pl.when(steps > 0)
            def _():
                start_fetch(i, 0, 0)

            def step_fn(s, carry):
                m, l, acc = carry
                buf = lax.rem(s, 2)

                @pl.when(s + 1 < steps)
                def _():
                    start_fetch(i, s + 1, lax.rem(s + 1, 2))

                wait_fetch(buf)
                kc = kbuf[buf].reshape(CHUNK * BS, H, D)
                vc = vbuf[buf].reshape(CHUNK * BS, H, D)
                sc = jnp.einsum("hd,khd->hk", qi, kc,
                                preferred_element_type=jnp.float32) * SCALE
                tok = lax.broadcasted_iota(jnp.int32, (1, CHUNK * BS), 1)
                key_ok = (s * CHUNK + tok // BS) < n
                sc = jnp.where(key_ok, sc, NEG)
                m_new = jnp.maximum(m, jnp.max(sc, axis=-1, keepdims=True))
                p = jnp.exp(sc - m_new)
                corr = jnp.exp(m - m_new)
                l_new = l * corr + jnp.sum(p, axis=-1, keepdims=True)
                pv = jnp.einsum("hk,khd->hd", p, vc,
                                preferred_element_type=jnp.float32)
                acc_new = acc * corr + pv
                return m_new, l_new, acc_new

            m0 = jnp.full((H, 1), NEG, dtype=jnp.float32)
            l0 = jnp.zeros((H, 1), dtype=jnp.float32)
            a0 = jnp.zeros((H, D), dtype=jnp.float32)
            m, l, acc = lax.fori_loop(0, steps, step_fn, (m0, l0, a0))

            acc_s[i] = acc
            ml_s[0, i] = m[:, 0]
            ml_s[1, i] = l[:, 0]

        partners = [
            (my_x, my_y, 1 - mz),
            (my_x, 1 - my_y, mz),
            (1 - my_x, my_y, mz),
        ]
        barrier = pltpu.get_barrier_semaphore()
        for p in partners:
            pl.semaphore_signal(barrier, inc=1, device_id=p,
                                device_id_type=pl.DeviceIdType.MESH)
        pl.semaphore_wait(barrier, len(partners))

        for s, p in enumerate(partners):
            r_acc = pltpu.make_async_remote_copy(
                src_ref=acc_s, dst_ref=racc.at[s], send_sem=ssem.at[s, 0],
                recv_sem=rsem.at[s, 0], device_id=p,
                device_id_type=pl.DeviceIdType.MESH)
            r_ml = pltpu.make_async_remote_copy(
                src_ref=ml_s, dst_ref=rml.at[s], send_sem=ssem.at[s, 1],
                recv_sem=rsem.at[s, 1], device_id=p,
                device_id_type=pl.DeviceIdType.MESH)
            r_acc.start()
            r_ml.start()
            r_acc.wait()
            r_ml.wait()

            m_a, l_a = ml_s[0], ml_s[1]
            m_b, l_b = rml[s, 0], rml[s, 1]
            m = jnp.maximum(m_a, m_b)
            wa = jnp.exp(m_a - m)
            wb = jnp.exp(m_b - m)
            l = wa * l_a + wb * l_b
            acc = (wa[:, :, None] * acc_s[...]
                   + wb[:, :, None] * racc[s])
            if s < 2:
                acc_s[...] = acc
                ml_s[0] = m
                ml_s[1] = l
            else:
                out_ref[...] = (acc / l[:, :, None]).reshape(B, 1, H, D)

    return pl.pallas_call(
        body,
        out_shape=jax.ShapeDtypeStruct((B, 1, H, D), jnp.float32),
        in_specs=[
            pl.BlockSpec(memory_space=pltpu.VMEM),
            pl.BlockSpec(memory_space=pltpu.SMEM),
            pl.BlockSpec(memory_space=pltpu.SMEM),
            pl.BlockSpec(memory_space=pl.ANY),
            pl.BlockSpec(memory_space=pl.ANY),
        ],
        out_specs=pl.BlockSpec(memory_space=pltpu.VMEM),
        scratch_shapes=[
            pltpu.VMEM((2, CHUNK, BS, H, D), jnp.float32),
            pltpu.VMEM((2, CHUNK, BS, H, D), jnp.float32),
            pltpu.SemaphoreType.DMA((2, CHUNK)),
            pltpu.SemaphoreType.DMA((2, CHUNK)),
            pltpu.VMEM((B, H, D), jnp.float32),
            pltpu.VMEM((2, B, H), jnp.float32),
            pltpu.VMEM((3, B, H, D), jnp.float32),
            pltpu.VMEM((3, 2, B, H), jnp.float32),
            pltpu.SemaphoreType.DMA((3, 2)),
            pltpu.SemaphoreType.DMA((3, 2)),
        ],
        compiler_params=pltpu.CompilerParams(collective_id=0),
    )(q, pages, counts, K, V)


# device time: 180309 ns/iter; 2.9329x vs baseline; 1.0125x over previous
import jax
import jax.numpy as jnp
from jax import lax
from jax.experimental import pallas as pl
from jax.experimental.pallas import tpu as pltpu

B, H, D, BS = 8, 8, 128, 16
NB = 512
NLOCAL = 512
CHUNK = 8
SCALE = D ** -0.5
NEG = -1e30


def kernel(Q, K, V, bt, lens):
    q = Q[:, 0]

    def body(q_ref, bt_ref, lens_ref, k_hbm, v_hbm, out_ref,
             pages_s, kbuf, vbuf, ksem, vsem,
             acc_s, ml_s, racc, rml, ssem, rsem):
        my_x = lax.axis_index("x")
        my_y = lax.axis_index("y")
        mz = lax.axis_index("z")
        my_q = my_x * 2 + my_y

        def start_fetch(i, step, buf):
            for c in range(CHUNK):
                p = pages_s[i, step * CHUNK + c] & (NLOCAL - 1)
                pltpu.make_async_copy(
                    k_hbm.at[p], kbuf.at[buf, c], ksem.at[buf, c]).start()
                pltpu.make_async_copy(
                    v_hbm.at[p], vbuf.at[buf, c], vsem.at[buf, c]).start()

        def wait_fetch(buf):
            for c in range(CHUNK):
                pltpu.make_async_copy(
                    k_hbm.at[0], kbuf.at[buf, c], ksem.at[buf, c]).wait()
                pltpu.make_async_copy(
                    v_hbm.at[0], vbuf.at[buf, c], vsem.at[buf, c]).wait()

        for i in range(B):
            nt = (lens_ref[i] - my_q + 3) // 4

            def comp_body(t, cnt):
                pv = bt_ref[i, my_q + 4 * t]
                ok = (pv // NLOCAL) == mz

                @pl.when(ok)
                def _():
                    pages_s[i, cnt] = pv & (NLOCAL - 1)

                return cnt + jnp.where(ok, 1, 0)

            n = lax.fori_loop(0, nt, comp_body, jnp.int32(0))
            steps = (n + CHUNK - 1) // CHUNK
            qi = q_ref[i]

            @pl.when(steps > 0)
            def _():
                start_fetch(i, 0, 0)

            def step_fn(s, carry):
                m, l, acc = carry
                buf = lax.rem(s, 2)

                @pl.when(s + 1 < steps)
                def _():
                    start_fetch(i, s + 1, lax.rem(s + 1, 2))

                wait_fetch(buf)
                kc = kbuf[buf].reshape(CHUNK * BS, H, D)
                vc = vbuf[buf].reshape(CHUNK * BS, H, D)
                sc = jnp.einsum("hd,khd->hk", qi, kc,
                                preferred_element_type=jnp.float32) * SCALE
                tok = lax.broadcasted_iota(jnp.int32, (1, CHUNK * BS), 1)
                key_ok = (s * CHUNK + tok // BS) < n
                sc = jnp.where(key_ok, sc, NEG)
                m_new = jnp.maximum(m, jnp.max(sc, axis=-1, keepdims=True))
                p = jnp.exp(sc - m_new)
                corr = jnp.exp(m - m_new)
                l_new = l * corr + jnp.sum(p, axis=-1, keepdims=True)
                pv = jnp.einsum("hk,khd->hd", p, vc,
                                preferred_element_type=jnp.float32)
                acc_new = acc * corr + pv
                return m_new, l_new, acc_new

            m0 = jnp.full((H, 1), NEG, dtype=jnp.float32)
            l0 = jnp.zeros((H, 1), dtype=jnp.float32)
            a0 = jnp.zeros((H, D), dtype=jnp.float32)
            m, l, acc = lax.fori_loop(0, steps, step_fn, (m0, l0, a0))

            acc_s[i] = acc
            ml_s[0, i] = m[:, 0]
            ml_s[1, i] = l[:, 0]

        partners = [
            (my_x, my_y, 1 - mz),
            (my_x, 1 - my_y, mz),
            (1 - my_x, my_y, mz),
        ]
        barrier = pltpu.get_barrier_semaphore()
        for p in partners:
            pl.semaphore_signal(barrier, inc=1, device_id=p,
                                device_id_type=pl.DeviceIdType.MESH)
        pl.semaphore_wait(barrier, len(partners))

        for s, p in enumerate(partners):
            r_acc = pltpu.make_async_remote_copy(
                src_ref=acc_s, dst_ref=racc.at[s], send_sem=ssem.at[s, 0],
                recv_sem=rsem.at[s, 0], device_id=p,
                device_id_type=pl.DeviceIdType.MESH)
            r_ml = pltpu.make_async_remote_copy(
                src_ref=ml_s, dst_ref=rml.at[s], send_sem=ssem.at[s, 1],
                recv_sem=rsem.at[s, 1], device_id=p,
                device_id_type=pl.DeviceIdType.MESH)
            r_acc.start()
            r_ml.start()
            r_acc.wait()
            r_ml.wait()

            m_a, l_a = ml_s[0], ml_s[1]
            m_b, l_b = rml[s, 0], rml[s, 1]
            m = jnp.maximum(m_a, m_b)
            wa = jnp.exp(m_a - m)
            wb = jnp.exp(m_b - m)
            l = wa * l_a + wb * l_b
            acc = (wa[:, :, None] * acc_s[...]
                   + wb[:, :, None] * racc[s])
            if s < 2:
                acc_s[...] = acc
                ml_s[0] = m
                ml_s[1] = l
            else:
                out_ref[...] = (acc / l[:, :, None]).reshape(B, 1, H, D)

    return pl.pallas_call(
        body,
        out_shape=jax.ShapeDtypeStruct((B, 1, H, D), jnp.float32),
        in_specs=[
            pl.BlockSpec(memory_space=pltpu.VMEM),
            pl.BlockSpec(memory_space=pltpu.SMEM),
            pl.BlockSpec(memory_space=pltpu.SMEM),
            pl.BlockSpec(memory_space=pl.ANY),
            pl.BlockSpec(memory_space=pl.ANY),
        ],
        out_specs=pl.BlockSpec(memory_space=pltpu.VMEM),
        scratch_shapes=[
            pltpu.SMEM((B, NB), jnp.int32),
            pltpu.VMEM((2, CHUNK, BS, H, D), jnp.float32),
            pltpu.VMEM((2, CHUNK, BS, H, D), jnp.float32),
            pltpu.SemaphoreType.DMA((2, CHUNK)),
            pltpu.SemaphoreType.DMA((2, CHUNK)),
            pltpu.VMEM((B, H, D), jnp.float32),
            pltpu.VMEM((2, B, H), jnp.float32),
            pltpu.VMEM((3, B, H, D), jnp.float32),
            pltpu.VMEM((3, 2, B, H), jnp.float32),
            pltpu.SemaphoreType.DMA((3, 2)),
            pltpu.SemaphoreType.DMA((3, 2)),
        ],
        compiler_params=pltpu.CompilerParams(collective_id=0),
    )(q, bt, lens, K, V)
